# Initial kernel scaffold; baseline (speedup 1.0000x reference)
import jax
import jax.numpy as jnp
from jax.experimental import pallas as pl


def kernel(x, W_route, b_route, W_noise, b_noise):
    raise NotImplementedError("write your pallas kernel here")


# speedup vs baseline: 3.3798x; 3.3798x over previous
import jax
import jax.numpy as jnp
from jax.experimental import pallas as pl
from jax.experimental.pallas import tpu as pltpu

_TOP_K = 8


def _router_block_kernel(x_ref, wrt_ref, wnt_ref, br_ref, bn_ref, g_ref,
                         out_ref, idx_ref):
    x = x_ref[...]
    logits = jnp.dot(x, wrt_ref[...], preferred_element_type=jnp.float32)
    logits = logits + br_ref[...]
    nlog = jnp.dot(x, wnt_ref[...], preferred_element_type=jnp.float32)
    nlog = nlog + bn_ref[...]
    noisy = logits + g_ref[...] * jax.nn.softplus(nlog)

    n_exp = noisy.shape[1]
    col = jax.lax.broadcasted_iota(jnp.int32, noisy.shape, 1)
    kcol = jax.lax.broadcasted_iota(jnp.int32, idx_ref.shape, 1)
    neg_inf = jnp.float32(-jnp.inf)

    work = noisy
    idx_out = jnp.zeros(idx_ref.shape, jnp.int32)
    for k in range(_TOP_K):
        m = jnp.max(work, axis=1, keepdims=True)
        amax = jnp.min(jnp.where(work == m, col, n_exp), axis=1, keepdims=True)
        idx_out = jnp.where(kcol == k, amax, idx_out)
        work = jnp.where(col == amax, neg_inf, work)
    idx_ref[...] = idx_out

    mask = work == neg_inf
    m0 = jnp.max(noisy, axis=1, keepdims=True)
    e = jnp.where(mask, jnp.exp(noisy - m0), 0.0)
    out_ref[...] = e / jnp.sum(e, axis=1, keepdims=True)


def kernel(x, W_route, b_route, W_noise, b_noise):
    n_tokens, d_model = x.shape
    n_experts = W_route.shape[0]
    blk = 512 if n_tokens % 512 == 0 else n_tokens
    grid = (n_tokens // blk,)

    gauss = jax.random.normal(jax.random.key(42), (n_tokens, n_experts),
                              dtype=jnp.float32)
    wrt = W_route.T
    wnt = W_noise.T
    br = b_route.reshape(1, n_experts)
    bn = b_noise.reshape(1, n_experts)

    router, indices = pl.pallas_call(
        _router_block_kernel,
        grid=grid,
        in_specs=[
            pl.BlockSpec((blk, d_model), lambda i: (i, 0)),
            pl.BlockSpec((d_model, n_experts), lambda i: (0, 0)),
            pl.BlockSpec((d_model, n_experts), lambda i: (0, 0)),
            pl.BlockSpec((1, n_experts), lambda i: (0, 0)),
            pl.BlockSpec((1, n_experts), lambda i: (0, 0)),
            pl.BlockSpec((blk, n_experts), lambda i: (i, 0)),
        ],
        out_specs=[
            pl.BlockSpec((blk, n_experts), lambda i: (i, 0)),
            pl.BlockSpec((blk, _TOP_K), lambda i: (i, 0)),
        ],
        out_shape=[
            jax.ShapeDtypeStruct((n_tokens, n_experts), jnp.float32),
            jax.ShapeDtypeStruct((n_tokens, _TOP_K), jnp.int32),
        ],
        compiler_params=pltpu.CompilerParams(
            dimension_semantics=("arbitrary",),
        ),
    )(x, wrt, wnt, br, bn, gauss)
    return (router, indices)


# speedup vs baseline: 4.0264x; 1.1913x over previous
import jax
import jax.numpy as jnp
from jax.experimental import pallas as pl
from jax.experimental.pallas import tpu as pltpu

_TOP_K = 8


def _router_block_kernel(x_ref, wcat_ref, bcat_ref, g_ref, out_ref, idx_ref):
    x = x_ref[...]
    acc = jnp.dot(x, wcat_ref[...], preferred_element_type=jnp.float32)
    acc = acc + bcat_ref[...]
    n_exp = acc.shape[1] // 2
    logits = acc[:, :n_exp]
    nlog = acc[:, n_exp:]
    noisy = logits + g_ref[...] * jax.nn.softplus(nlog)

    col = jax.lax.broadcasted_iota(jnp.int32, noisy.shape, 1)
    kcol = jax.lax.broadcasted_iota(jnp.int32, idx_ref.shape, 1)
    bits = pltpu.bitcast(noisy, jnp.int32)
    key = bits ^ (jnp.int32(0x7FFFFFFF) & (bits >> 31))
    key = (key & jnp.int32(~63)) | (jnp.int32(63) ^ col)
    sentinel = jnp.int32(-2147483648)

    work = key
    idx_out = jnp.zeros(idx_ref.shape, jnp.int32)
    for k in range(_TOP_K):
        m = jnp.max(work, axis=1, keepdims=True)
        amax = jnp.int32(63) ^ (m & jnp.int32(63))
        idx_out = jnp.where(kcol == k, amax, idx_out)
        work = jnp.where(work == m, sentinel, work)
    idx_ref[...] = idx_out

    mask = work == sentinel
    m0 = jnp.max(noisy, axis=1, keepdims=True)
    e = jnp.where(mask, jnp.exp(noisy - m0), 0.0)
    out_ref[...] = e / jnp.sum(e, axis=1, keepdims=True)


def kernel(x, W_route, b_route, W_noise, b_noise):
    n_tokens, d_model = x.shape
    n_experts = W_route.shape[0]
    blk = 512 if n_tokens % 512 == 0 else n_tokens
    grid = (n_tokens // blk,)

    gauss = jax.random.normal(jax.random.key(42), (n_tokens, n_experts),
                              dtype=jnp.float32)
    wcat = jnp.concatenate([W_route.T, W_noise.T], axis=1)
    bcat = jnp.concatenate([b_route, b_noise]).reshape(1, 2 * n_experts)

    router, indices = pl.pallas_call(
        _router_block_kernel,
        grid=grid,
        in_specs=[
            pl.BlockSpec((blk, d_model), lambda i: (i, 0)),
            pl.BlockSpec((d_model, 2 * n_experts), lambda i: (0, 0)),
            pl.BlockSpec((1, 2 * n_experts), lambda i: (0, 0)),
            pl.BlockSpec((blk, n_experts), lambda i: (i, 0)),
        ],
        out_specs=[
            pl.BlockSpec((blk, n_experts), lambda i: (i, 0)),
            pl.BlockSpec((blk, _TOP_K), lambda i: (i, 0)),
        ],
        out_shape=[
            jax.ShapeDtypeStruct((n_tokens, n_experts), jnp.float32),
            jax.ShapeDtypeStruct((n_tokens, _TOP_K), jnp.int32),
        ],
        compiler_params=pltpu.CompilerParams(
            dimension_semantics=("arbitrary",),
        ),
    )(x, wcat, bcat, gauss)
    return (router, indices)


# speedup vs baseline: 4.3217x; 1.0733x over previous
import jax
import jax.numpy as jnp
from jax.experimental import pallas as pl
from jax.experimental.pallas import tpu as pltpu

_TOP_K = 8


def _router_block_kernel(x_ref, wcat_ref, bcat_ref, g_ref, out_ref, idx_ref):
    x = x_ref[...]
    acc = jnp.dot(x, wcat_ref[...], preferred_element_type=jnp.float32)
    acc = acc + bcat_ref[...]
    n_exp = acc.shape[1] // 2
    logits = acc[:, :n_exp]
    nlog = acc[:, n_exp:]
    noisy = logits + g_ref[...] * jax.nn.softplus(nlog)

    rows = noisy.shape[0]
    chunk = 64 if rows % 64 == 0 else rows
    neg_inf = jnp.float32(-jnp.inf)
    for c in range(rows // chunk):
        nz = noisy[c * chunk:(c + 1) * chunk, :]
        col = jax.lax.broadcasted_iota(jnp.int32, nz.shape, 1)
        bits = pltpu.bitcast(nz, jnp.int32)
        idxbits = jnp.where(bits < 0, col, jnp.int32(63) ^ col)
        work = pltpu.bitcast((bits & jnp.int32(~63)) | idxbits, jnp.float32)
        kcol = jax.lax.broadcasted_iota(jnp.int32, (chunk, _TOP_K), 1)
        idx_out = jnp.zeros((chunk, _TOP_K), jnp.int32)
        m0 = None
        for k in range(_TOP_K):
            m = jnp.max(work, axis=1, keepdims=True)
            mb = pltpu.bitcast(m, jnp.int32)
            low6 = mb & jnp.int32(63)
            amax = jnp.where(mb < 0, low6, jnp.int32(63) ^ low6)
            idx_out = jnp.where(kcol == k, amax, idx_out)
            if k == 0:
                m0 = m
            work = jnp.where(work == m, neg_inf, work)
        idx_ref[c * chunk:(c + 1) * chunk, :] = idx_out

        mask = work == neg_inf
        e = jnp.where(mask, jnp.exp(nz - m0), 0.0)
        out_ref[c * chunk:(c + 1) * chunk, :] = e / jnp.sum(e, axis=1, keepdims=True)


def kernel(x, W_route, b_route, W_noise, b_noise):
    n_tokens, d_model = x.shape
    n_experts = W_route.shape[0]
    blk = 512 if n_tokens % 512 == 0 else n_tokens
    grid = (n_tokens // blk,)

    gauss = jax.random.normal(jax.random.key(42), (n_tokens, n_experts),
                              dtype=jnp.float32)
    wcat = jnp.concatenate([W_route.T, W_noise.T], axis=1)
    bcat = jnp.concatenate([b_route, b_noise]).reshape(1, 2 * n_experts)

    router, indices = pl.pallas_call(
        _router_block_kernel,
        grid=grid,
        in_specs=[
            pl.BlockSpec((blk, d_model), lambda i: (i, 0)),
            pl.BlockSpec((d_model, 2 * n_experts), lambda i: (0, 0)),
            pl.BlockSpec((1, 2 * n_experts), lambda i: (0, 0)),
            pl.BlockSpec((blk, n_experts), lambda i: (i, 0)),
        ],
        out_specs=[
            pl.BlockSpec((blk, n_experts), lambda i: (i, 0)),
            pl.BlockSpec((blk, _TOP_K), lambda i: (i, 0)),
        ],
        out_shape=[
            jax.ShapeDtypeStruct((n_tokens, n_experts), jnp.float32),
            jax.ShapeDtypeStruct((n_tokens, _TOP_K), jnp.int32),
        ],
        compiler_params=pltpu.CompilerParams(
            dimension_semantics=("arbitrary",),
        ),
    )(x, wcat, bcat, gauss)
    return (router, indices)


# speedup vs baseline: 5.2928x; 1.2247x over previous
import jax
import jax.numpy as jnp
from jax.experimental import pallas as pl
from jax.experimental.pallas import tpu as pltpu

_TOP_K = 8

_ROT = ((13, 15, 26, 6), (17, 29, 16, 24))


def _gauss_block(flat_u32):
    k0 = jnp.uint32(0)
    k1 = jnp.uint32(42)
    ks = (k0, k1, k0 ^ k1 ^ jnp.uint32(0x1BD11BDA))
    x0 = jnp.zeros_like(flat_u32) + ks[0]
    x1 = flat_u32 + ks[1]
    for i in range(5):
        for r in _ROT[i % 2]:
            x0 = x0 + x1
            x1 = (x1 << jnp.uint32(r)) | (x1 >> jnp.uint32(32 - r))
            x1 = x1 ^ x0
        x0 = x0 + ks[(i + 1) % 3]
        x1 = x1 + ks[(i + 2) % 3] + jnp.uint32(i + 1)
    bits = x0 ^ x1

    u = pltpu.bitcast((bits >> jnp.uint32(9)) | jnp.uint32(0x3F800000),
                      jnp.float32) - 1.0
    lo = jnp.float32(-0.99999994)
    x = u * (jnp.float32(1.0) - lo) + lo
    w = -jnp.log1p(-x * x)
    small = w < 5.0
    ws = w - 2.5
    wl = jnp.sqrt(w) - 3.0
    p1 = jnp.full_like(x, 2.81022636e-08)
    for c in (3.43273939e-07, -3.5233877e-06, -4.39150654e-06, 0.00021858087,
              -0.00125372503, -0.00417768164, 0.246640727, 1.50140941):
        p1 = p1 * ws + jnp.float32(c)
    p2 = jnp.full_like(x, -0.000200214257)
    for c in (0.000100950558, 0.00134934322, -0.00367342844, 0.00573950773,
              -0.0076224613, 0.00943887047, 1.00167406, 2.83297682):
        p2 = p2 * wl + jnp.float32(c)
    p = jnp.where(small, p1, p2)
    return jnp.float32(1.4142135623730951) * (p * x)


def _router_block_kernel(x_ref, wcat_ref, bcat_ref, out_ref, idx_ref):
    x = x_ref[...]
    acc = jnp.dot(x, wcat_ref[...], preferred_element_type=jnp.float32)
    acc = acc + bcat_ref[...]
    n_exp = acc.shape[1] // 2
    rows = acc.shape[0]
    logits = acc[:, :n_exp]
    nlog = acc[:, n_exp:]

    row_i = jax.lax.broadcasted_iota(jnp.int32, (rows, n_exp), 0)
    col_i = jax.lax.broadcasted_iota(jnp.int32, (rows, n_exp), 1)
    base = pl.program_id(0) * (rows * n_exp)
    flat = (base + row_i * n_exp + col_i).astype(jnp.uint32)
    gauss = _gauss_block(flat)

    noisy = logits + gauss * jax.nn.softplus(nlog)

    chunk = 64 if rows % 64 == 0 else rows
    neg_inf = jnp.float32(-jnp.inf)
    for c in range(rows // chunk):
        nz = noisy[c * chunk:(c + 1) * chunk, :]
        col = jax.lax.broadcasted_iota(jnp.int32, nz.shape, 1)
        bits = pltpu.bitcast(nz, jnp.int32)
        idxbits = jnp.where(bits < 0, col, jnp.int32(63) ^ col)
        work = pltpu.bitcast((bits & jnp.int32(~63)) | idxbits, jnp.float32)
        kcol = jax.lax.broadcasted_iota(jnp.int32, (chunk, _TOP_K), 1)
        idx_out = jnp.zeros((chunk, _TOP_K), jnp.int32)
        m0 = None
        for k in range(_TOP_K):
            m = jnp.max(work, axis=1, keepdims=True)
            mb = pltpu.bitcast(m, jnp.int32)
            low6 = mb & jnp.int32(63)
            amax = jnp.where(mb < 0, low6, jnp.int32(63) ^ low6)
            idx_out = jnp.where(kcol == k, amax, idx_out)
            if k == 0:
                m0 = m
            work = jnp.where(work == m, neg_inf, work)
        idx_ref[c * chunk:(c + 1) * chunk, :] = idx_out

        mask = work == neg_inf
        e = jnp.where(mask, jnp.exp(nz - m0), 0.0)
        out_ref[c * chunk:(c + 1) * chunk, :] = e / jnp.sum(e, axis=1, keepdims=True)


def kernel(x, W_route, b_route, W_noise, b_noise):
    n_tokens, d_model = x.shape
    n_experts = W_route.shape[0]
    blk = 512 if n_tokens % 512 == 0 else n_tokens
    grid = (n_tokens // blk,)

    wcat = jnp.concatenate([W_route.T, W_noise.T], axis=1)
    bcat = jnp.concatenate([b_route, b_noise]).reshape(1, 2 * n_experts)

    router, indices = pl.pallas_call(
        _router_block_kernel,
        grid=grid,
        in_specs=[
            pl.BlockSpec((blk, d_model), lambda i: (i, 0)),
            pl.BlockSpec((d_model, 2 * n_experts), lambda i: (0, 0)),
            pl.BlockSpec((1, 2 * n_experts), lambda i: (0, 0)),
        ],
        out_specs=[
            pl.BlockSpec((blk, n_experts), lambda i: (i, 0)),
            pl.BlockSpec((blk, _TOP_K), lambda i: (i, 0)),
        ],
        out_shape=[
            jax.ShapeDtypeStruct((n_tokens, n_experts), jnp.float32),
            jax.ShapeDtypeStruct((n_tokens, _TOP_K), jnp.int32),
        ],
        compiler_params=pltpu.CompilerParams(
            dimension_semantics=("arbitrary",),
        ),
    )(x, wcat, bcat)
    return (router, indices)
